# Initial kernel scaffold; baseline (speedup 1.0000x reference)
import jax
import jax.numpy as jnp
from jax.experimental import pallas as pl


def kernel(logits, targets):
    raise NotImplementedError("write your pallas kernel here")


# speedup vs baseline: 3.2250x; 3.2250x over previous
import functools
import jax
import jax.numpy as jnp
from jax import lax
from jax.experimental import pallas as pl
from jax.experimental.pallas import tpu as pltpu
from jax.experimental.pallas import tpu_sc as plsc

B = 4096
N = 1000
K = 9
NC = 2
NS = 16
NW = NC * NS
ROWS_PER_W = B // NW
G = 16
NG = ROWS_PER_W // G
UNROLL = 4
LN2 = 0.6931471805599453


def _log_1_to_16(s):
    bits = lax.bitcast_convert_type(s, jnp.int32)
    e = jnp.float32(1.0) * ((bits >> 23) - 127)
    m = lax.bitcast_convert_type(
        (bits & jnp.int32(0x007FFFFF)) | jnp.int32(0x3F800000), jnp.float32)
    u = (m - 1.0) / (m + 1.0)
    u2 = u * u
    p = 2.0 * u * (1.0 + u2 * (1.0 / 3.0 + u2 * (1.0 / 5.0
                   + u2 * (1.0 / 7.0 + u2 * (1.0 / 9.0)))))
    return e * LN2 + p


def _mmcl_body(lg_hbm, tg_hbm, out_hbm, buf0, buf1, tgts, ovec, sem0, sem1):
    wid = lax.axis_index("s") * NC + lax.axis_index("c")
    row0 = wid * ROWS_PER_W
    lanes = lax.iota(jnp.int32, 16)

    pltpu.sync_copy(tg_hbm.at[pl.ds(row0 * 1, ROWS_PER_W)], tgts)

    sems = [sem0, sem1]
    bufs = [buf0, buf1]
    acc = jnp.zeros((16,), jnp.float32)
    neg_inf = jnp.full((16,), -jnp.inf, jnp.float32)

    pending = pltpu.async_copy(
        lg_hbm.at[pl.ds(row0 * N, G * N)], bufs[0], sems[0])
    for g in range(NG):
        cur = g % 2
        nxt = (g + 1) % 2
        pending.wait()
        if g + 1 < NG:
            pending = pltpu.async_copy(
                lg_hbm.at[pl.ds((row0 + (g + 1) * G) * N, G * N)],
                bufs[nxt], sems[nxt])

        bufv = bufs[cur]
        tgt16 = tgts[pl.ds(g * G, 16)]
        flatpos = lanes * N + tgt16

        idx0 = lanes * N
        ts0 = tuple(neg_inf for _ in range(K))

        def body(i, carry, bufv=bufv, flatpos=flatpos):
            idx, ts = carry
            for _ in range(UNROLL):
                v = plsc.load_gather(bufv, [idx])
                v = jnp.where(idx == flatpos, neg_inf, v)
                new = v
                ts2 = []
                for t in ts:
                    hi = jnp.maximum(t, new)
                    lo = jnp.minimum(t, new)
                    ts2.append(hi)
                    new = lo
                ts = tuple(ts2)
                idx = idx + 1
            return idx, ts

        _, ts = lax.fori_loop(0, N // UNROLL, body, (idx0, ts0))

        pos = plsc.load_gather(bufv, [flatpos])
        posx = pos * 10.0
        mx = jnp.maximum(ts[0] * 10.0, posx)
        s = 2.0 * jnp.exp(posx - mx)
        for t in ts:
            s = s + jnp.exp(t * 10.0 - mx)
        loss = _log_1_to_16(s) + mx - posx
        acc = acc + loss * (1.0 / B)

    ovec[...] = acc
    pltpu.sync_copy(ovec, out_hbm.at[pl.ds(wid * 16, 16)])


@jax.jit
def _mmcl(logits_flat, targets):
    mesh = plsc.VectorSubcoreMesh(core_axis_name="c", subcore_axis_name="s")
    partials = pl.kernel(
        _mmcl_body,
        mesh=mesh,
        compiler_params=pltpu.CompilerParams(needs_layout_passes=False),
        out_type=jax.ShapeDtypeStruct((NW * 16,), jnp.float32),
        scratch_types=[
            pltpu.VMEM((G * N,), jnp.float32),
            pltpu.VMEM((G * N,), jnp.float32),
            pltpu.VMEM((ROWS_PER_W,), jnp.int32),
            pltpu.VMEM((16,), jnp.float32),
            pltpu.SemaphoreType.DMA,
            pltpu.SemaphoreType.DMA,
        ],
    )(logits_flat, targets)
    return jnp.sum(partials)


def kernel(logits, targets):
    logits_flat = jnp.reshape(logits, (-1,))
    targets = targets.astype(jnp.int32)
    return _mmcl(logits_flat, targets)


# speedup vs baseline: 3.3731x; 1.0459x over previous
import functools
import jax
import jax.numpy as jnp
from jax import lax
from jax.experimental import pallas as pl
from jax.experimental.pallas import tpu as pltpu
from jax.experimental.pallas import tpu_sc as plsc

B = 4096
N = 1000
K = 9
NC = 2
NS = 16
NW = NC * NS
ROWS_PER_W = B // NW
G = 16
NG = ROWS_PER_W // G
UNROLL = 8
LN2 = 0.6931471805599453


def _log_1_to_16(s):
    bits = lax.bitcast_convert_type(s, jnp.int32)
    e = jnp.float32(1.0) * ((bits >> 23) - 127)
    m = lax.bitcast_convert_type(
        (bits & jnp.int32(0x007FFFFF)) | jnp.int32(0x3F800000), jnp.float32)
    u = (m - 1.0) / (m + 1.0)
    u2 = u * u
    p = 2.0 * u * (1.0 + u2 * (1.0 / 3.0 + u2 * (1.0 / 5.0
                   + u2 * (1.0 / 7.0 + u2 * (1.0 / 9.0)))))
    return e * LN2 + p


def _mmcl_body(lg_hbm, tg_hbm, out_hbm, buf0, buf1, tgts, ovec, sem0, sem1):
    wid = lax.axis_index("s") * NC + lax.axis_index("c")
    row0 = wid * ROWS_PER_W
    lanes = lax.iota(jnp.int32, 16)

    pltpu.sync_copy(tg_hbm.at[pl.ds(row0 * 1, ROWS_PER_W)], tgts)

    sems = [sem0, sem1]
    bufs = [buf0, buf1]
    acc = jnp.zeros((16,), jnp.float32)
    neg_inf = jnp.full((16,), -jnp.inf, jnp.float32)

    pending = pltpu.async_copy(
        lg_hbm.at[pl.ds(row0 * N, G * N)], bufs[0], sems[0])
    for g in range(NG):
        cur = g % 2
        nxt = (g + 1) % 2
        pending.wait()
        if g + 1 < NG:
            pending = pltpu.async_copy(
                lg_hbm.at[pl.ds((row0 + (g + 1) * G) * N, G * N)],
                bufs[nxt], sems[nxt])

        bufv = bufs[cur]
        tgt16 = tgts[pl.ds(g * G, 16)]
        flatpos = lanes * N + tgt16

        pos = plsc.load_gather(bufv, [flatpos])
        plsc.store_scatter(bufv, [flatpos], neg_inf)

        idx0 = lanes * N
        ts0 = tuple(neg_inf for _ in range(K))

        def body(i, carry, bufv=bufv):
            idx, ts = carry
            for _ in range(UNROLL):
                v = plsc.load_gather(bufv, [idx])
                new = v
                ts2 = []
                for t in ts:
                    hi = jnp.maximum(t, new)
                    lo = jnp.minimum(t, new)
                    ts2.append(hi)
                    new = lo
                ts = tuple(ts2)
                idx = idx + 1
            return idx, ts

        _, ts = lax.fori_loop(0, N // UNROLL, body, (idx0, ts0))

        posx = pos * 10.0
        mx = jnp.maximum(ts[0] * 10.0, posx)
        s = 2.0 * jnp.exp(posx - mx)
        for t in ts:
            s = s + jnp.exp(t * 10.0 - mx)
        loss = _log_1_to_16(s) + mx - posx
        acc = acc + loss * (1.0 / B)

    ovec[...] = acc
    pltpu.sync_copy(ovec, out_hbm.at[pl.ds(wid * 16, 16)])


@jax.jit
def _mmcl(logits_flat, targets):
    mesh = plsc.VectorSubcoreMesh(core_axis_name="c", subcore_axis_name="s")
    partials = pl.kernel(
        _mmcl_body,
        mesh=mesh,
        compiler_params=pltpu.CompilerParams(needs_layout_passes=False),
        out_type=jax.ShapeDtypeStruct((NW * 16,), jnp.float32),
        scratch_types=[
            pltpu.VMEM((G * N,), jnp.float32),
            pltpu.VMEM((G * N,), jnp.float32),
            pltpu.VMEM((ROWS_PER_W,), jnp.int32),
            pltpu.VMEM((16,), jnp.float32),
            pltpu.SemaphoreType.DMA,
            pltpu.SemaphoreType.DMA,
        ],
    )(logits_flat, targets)
    return jnp.sum(partials)


def kernel(logits, targets):
    logits_flat = jnp.reshape(logits, (-1,))
    targets = targets.astype(jnp.int32)
    return _mmcl(logits_flat, targets)
